# Initial kernel scaffold; baseline (speedup 1.0000x reference)
import jax
import jax.numpy as jnp
from jax.experimental import pallas as pl


def kernel(inputs, id_table, pos_table):
    raise NotImplementedError("write your pallas kernel here")


# speedup vs baseline: 3.0952x; 3.0952x over previous
import functools

import jax
import jax.numpy as jnp
from jax import lax
from jax.experimental import pallas as pl
from jax.experimental.pallas import tpu as pltpu
from jax.experimental.pallas import tpu_sc as plsc

NC, NS = 2, 16
NW = NC * NS
GROUP = 128
LANES = 16


def _sc_embed(table, idx3d, pos_table):
    _, groups_per_w, _ = idx3d.shape
    D = table.shape[1]
    S = pos_table.shape[0]
    total_rows = NW * groups_per_w * GROUP
    mesh = plsc.VectorSubcoreMesh(core_axis_name="c", subcore_axis_name="s")

    @functools.partial(
        pl.kernel,
        out_type=jax.ShapeDtypeStruct((total_rows, D), jnp.float32),
        mesh=mesh,
        scratch_types=[
            pltpu.VMEM((groups_per_w, GROUP), jnp.int32),
            pltpu.VMEM((S, D), jnp.float32),
            pltpu.VMEM((GROUP, D), jnp.float32),
            pltpu.SemaphoreType.DMA,
        ],
        compiler_params=pltpu.CompilerParams(use_tc_tiling_on_sc=False),
    )
    def k(table_hbm, idx_hbm, pos_hbm, out_hbm, idx_v, pos_v, buf, sem):
        wid = lax.axis_index("s") * NC + lax.axis_index("c")
        pltpu.sync_copy(idx_hbm.at[wid], idx_v)
        pltpu.sync_copy(pos_hbm, pos_v)

        def group_body(g, carry):
            pltpu.async_copy(table_hbm.at[idx_v.at[g]], buf, sem).wait()
            base_r = lax.rem(g * GROUP, S)

            def row_body(r, c):
                pr = lax.rem(base_r + r, S)
                for j in range(D // LANES):
                    sl = pl.ds(j * LANES, LANES)
                    plsc.addupdate(buf.at[r, sl], pos_v[pr, sl])
                return c

            lax.fori_loop(0, GROUP, row_body, 0)
            pltpu.sync_copy(
                buf, out_hbm.at[pl.ds((wid * groups_per_w + g) * GROUP, GROUP)])
            return carry

        lax.fori_loop(0, groups_per_w, group_body, 0)

    return k(table, idx3d, pos_table)


def kernel(inputs, id_table, pos_table):
    B, S = inputs.shape
    V, D = id_table.shape
    flat = B * S
    groups_per_w = flat // (NW * GROUP)
    idx3d = inputs.reshape(NW, groups_per_w, GROUP).astype(jnp.int32)
    out = _sc_embed(id_table, idx3d, pos_table)
    return out.reshape(B, S, D)


# speedup vs baseline: 5.0632x; 1.6358x over previous
import functools

import jax
import jax.numpy as jnp
from jax import lax
from jax.experimental import pallas as pl
from jax.experimental.pallas import tpu as pltpu
from jax.experimental.pallas import tpu_sc as plsc

NC, NS = 2, 16
NW = NC * NS
GROUP = 128
LANES = 16


def _sc_embed(table, idx3d, pos_ext, seq_len):
    _, groups_per_w, _ = idx3d.shape
    D = table.shape[1]
    total_rows = NW * groups_per_w * GROUP
    mesh = plsc.VectorSubcoreMesh(core_axis_name="c", subcore_axis_name="s")

    @functools.partial(
        pl.kernel,
        out_type=jax.ShapeDtypeStruct((total_rows, D), jnp.float32),
        mesh=mesh,
        scratch_types=[
            pltpu.VMEM((groups_per_w, GROUP), jnp.int32),
            pltpu.VMEM(pos_ext.shape, jnp.float32),
            pltpu.VMEM((2, GROUP, D), jnp.float32),
            pltpu.VMEM((GROUP, D), jnp.float32),
            pltpu.SemaphoreType.DMA,
            pltpu.SemaphoreType.DMA,
        ],
        compiler_params=pltpu.CompilerParams(use_tc_tiling_on_sc=False),
    )
    def k(table_hbm, idx_hbm, pos_hbm, out_hbm, idx_v, pos_v, gbufs, obuf,
          gsem0, gsem1):
        wid = lax.axis_index("s") * NC + lax.axis_index("c")
        pltpu.sync_copy(idx_hbm.at[wid], idx_v)
        pltpu.sync_copy(pos_hbm, pos_v)
        gsems = (gsem0, gsem1)

        for b in range(2):
            pltpu.async_copy(table_hbm.at[idx_v.at[b]], gbufs.at[b], gsems[b])

        def step_body(s, carry):
            for b in range(2):
                g = 2 * s + b
                gb = gbufs.at[b]
                pltpu.make_async_copy(
                    table_hbm.at[idx_v.at[g]], gb, gsems[b]).wait()
                base_r = lax.rem(g * GROUP, seq_len)

                @plsc.parallel_loop(0, GROUP, 1, unroll=4)
                def add_body(r):
                    pr = base_r + r
                    for j in range(D // LANES):
                        sl = pl.ds(j * LANES, LANES)
                        obuf[r, sl] = gb[r, sl] + pos_v[pr, sl]

                @pl.when(g + 2 < groups_per_w)
                def _():
                    pltpu.async_copy(
                        table_hbm.at[idx_v.at[g + 2]], gb, gsems[b])

                pltpu.sync_copy(
                    obuf,
                    out_hbm.at[pl.ds((wid * groups_per_w + g) * GROUP, GROUP)])
            return carry

        lax.fori_loop(0, groups_per_w // 2, step_body, 0)

    return k(table, idx3d, pos_ext)


def kernel(inputs, id_table, pos_table):
    B, S = inputs.shape
    V, D = id_table.shape
    flat = B * S
    groups_per_w = flat // (NW * GROUP)
    idx3d = inputs.reshape(NW, groups_per_w, GROUP).astype(jnp.int32)
    n_ext = -(-(S - 1 + GROUP) // 8) * 8
    pos_ext = jnp.tile(pos_table, (-(-n_ext // S), 1))[:n_ext]
    out = _sc_embed(id_table, idx3d, pos_ext, S)
    return out.reshape(B, S, D)
